# Initial kernel scaffold; baseline (speedup 1.0000x reference)
import jax
import jax.numpy as jnp
from jax.experimental import pallas as pl


def kernel(x, edge_index, W1, b1, W2, b2):
    raise NotImplementedError("write your pallas kernel here")


# speedup vs baseline: 5.4688x; 5.4688x over previous
import functools

import jax
import jax.numpy as jnp
from jax import lax
from jax.experimental import pallas as pl
from jax.experimental.pallas import tpu as pltpu
from jax.experimental.pallas import tpu_sc as plsc

N = 10000
E = 320000
D = 128
NC, NS = 2, 16
NPAD = 10240
ROWS_PER_TILE = NPAD // NS
C = 128
CHUNKS = E // C
CHUNKS_PER_CORE = CHUNKS // NC
FULL_PER_TILE = CHUNKS_PER_CORE // NS
REM = CHUNKS_PER_CORE - FULL_PER_TILE * NS

_mesh = plsc.VectorSubcoreMesh(core_axis_name="c", subcore_axis_name="s")


@functools.partial(
    pl.kernel,
    mesh=_mesh,
    out_type=jax.ShapeDtypeStruct((NC, NPAD, D), jnp.float32),
    scratch_types=[
        pltpu.VMEM((C,), jnp.int32),
        pltpu.VMEM((C,), jnp.int32),
        pltpu.VMEM((C, D), jnp.float32),
        pltpu.VMEM_SHARED((NPAD, D), jnp.float32),
    ],
)
def _prop(h_hbm, src_hbm, dst_hbm, zeros_hbm, out_hbm,
          src_v, dst_v, rows_v, acc_sh):
    cid = lax.axis_index("c")
    sid = lax.axis_index("s")

    pltpu.sync_copy(zeros_hbm, acc_sh.at[pl.ds(sid * ROWS_PER_TILE,
                                               ROWS_PER_TILE)])
    plsc.subcore_barrier()

    def chunk_op(c):
        base = c * C
        pltpu.sync_copy(src_hbm.at[pl.ds(base, C)], src_v)
        pltpu.sync_copy(dst_hbm.at[pl.ds(base, C)], dst_v)
        pltpu.sync_copy(h_hbm.at[src_v], rows_v)
        pltpu.sync_copy(rows_v, acc_sh.at[dst_v], add=True)

    first = cid * CHUNKS_PER_CORE + sid * FULL_PER_TILE

    def body(i, carry):
        chunk_op(first + i)
        return carry

    lax.fori_loop(0, FULL_PER_TILE, body, 0)

    @pl.when(sid < REM)
    def _():
        chunk_op(cid * CHUNKS_PER_CORE + NS * FULL_PER_TILE + sid)

    plsc.subcore_barrier()

    sl = pl.ds(sid * ROWS_PER_TILE, ROWS_PER_TILE)
    pltpu.sync_copy(acc_sh.at[sl], out_hbm.at[cid, sl])


ROW_BLK = 1000
GRID = N // ROW_BLK

def _p_spec(which):
    return pl.BlockSpec((1, ROW_BLK, D), lambda i, w=which: (w, i, 0))

_x_spec = pl.BlockSpec((ROW_BLK, D), lambda i: (i, 0))


def _add3_body(p0_ref, p1_ref, x_ref, o_ref):
    o_ref[...] = p0_ref[0] + p1_ref[0] + x_ref[...]


def _add3(p, x):
    return pl.pallas_call(
        _add3_body,
        grid=(GRID,),
        in_specs=[_p_spec(0), _p_spec(1), _x_spec],
        out_specs=_x_spec,
        out_shape=jax.ShapeDtypeStruct((N, D), jnp.float32),
    )(p, p, x)


_SELU_ALPHA = 1.6732632423543772
_SELU_SCALE = 1.0507009873554805


def _mlp_body(p0_ref, p1_ref, h_ref, w_ref, b_ref, o_ref):
    h2 = p0_ref[0] + p1_ref[0] + h_ref[...]
    z = jnp.dot(h2, w_ref[...], preferred_element_type=jnp.float32)
    z = z + b_ref[...]
    o_ref[...] = _SELU_SCALE * jnp.where(
        z > 0, z, _SELU_ALPHA * (jnp.exp(z) - 1.0))


def _mlp(p, h, W1, b1):
    return pl.pallas_call(
        _mlp_body,
        grid=(GRID,),
        in_specs=[
            _p_spec(0), _p_spec(1), _x_spec,
            pl.BlockSpec((D, D), lambda i: (0, 0)),
            pl.BlockSpec((1, D), lambda i: (0, 0)),
        ],
        out_specs=_x_spec,
        out_shape=jax.ShapeDtypeStruct((N, D), jnp.float32),
    )(p, p, h, W1, b1.reshape(1, D))


def _out_body(p0_ref, p1_ref, h_ref, w_ref, b_ref, o_ref):
    g = p0_ref[0] + p1_ref[0] + h_ref[...]
    g = jnp.dot(g, w_ref[...], preferred_element_type=jnp.float32)
    g = g + b_ref[...]
    m = jnp.max(g, axis=1, keepdims=True)
    e = g - m
    lse = jnp.log(jnp.sum(jnp.exp(e), axis=1, keepdims=True))
    o_ref[...] = e - lse


def _outp(p, h, W2, b2):
    odim = W2.shape[1]
    return pl.pallas_call(
        _out_body,
        grid=(GRID,),
        in_specs=[
            _p_spec(0), _p_spec(1), _x_spec,
            pl.BlockSpec((D, odim), lambda i: (0, 0)),
            pl.BlockSpec((1, odim), lambda i: (0, 0)),
        ],
        out_specs=pl.BlockSpec((ROW_BLK, odim), lambda i: (i, 0)),
        out_shape=jax.ShapeDtypeStruct((N, odim), jnp.float32),
    )(p, p, h, W2, b2.reshape(1, odim))


def kernel(x, edge_index, W1, b1, W2, b2):
    src = edge_index[0].astype(jnp.int32)
    dst = edge_index[1].astype(jnp.int32)
    zeros = jnp.zeros((ROWS_PER_TILE, D), jnp.float32)

    p = _prop(x, src, dst, zeros)
    h1 = _add3(p, x)
    p = _prop(h1, src, dst, zeros)
    h = _mlp(p, h1, W1, b1)
    p = _prop(h, src, dst, zeros)
    return _outp(p, h, W2, b2)


# speedup vs baseline: 11.4227x; 2.0887x over previous
import functools

import jax
import jax.numpy as jnp
from jax import lax
from jax.experimental import pallas as pl
from jax.experimental.pallas import tpu as pltpu
from jax.experimental.pallas import tpu_sc as plsc

N = 10000
E = 320000
D = 128
NC, NS = 2, 16
ROWS_PER_TILE = 632
LAST_ROWS = N - 15 * ROWS_PER_TILE
C = 128
CHUNKS = E // C
CHUNKS_PER_CORE = CHUNKS // NC
FULL_PER_TILE = CHUNKS_PER_CORE // NS
REM = CHUNKS_PER_CORE - FULL_PER_TILE * NS
NIB = 4
NRB = 2

_mesh = plsc.VectorSubcoreMesh(core_axis_name="c", subcore_axis_name="s")

_full = jax.ShapeDtypeStruct((N, D), jnp.float32)


@functools.partial(
    pl.kernel,
    mesh=_mesh,
    out_type=(_full, _full),
    scratch_types=[
        pltpu.VMEM((NIB, C), jnp.int32),
        pltpu.VMEM((NIB, C), jnp.int32),
        pltpu.VMEM((NRB, C, D), jnp.float32),
        pltpu.VMEM_SHARED((N, D), jnp.float32),
        pltpu.SemaphoreType.DMA((NIB,)),
        pltpu.SemaphoreType.DMA((NRB,)),
        pltpu.SemaphoreType.DMA((NRB,)),
    ],
)
def _prop(h_hbm, src_hbm, dst_hbm, zeros_hbm, o0_hbm, o1_hbm,
          sidx_v, didx_v, rows_v, acc_sh, isem, gsem, ssem):
    cid = lax.axis_index("c")
    sid = lax.axis_index("s")

    n_i = FULL_PER_TILE + jnp.where(sid < REM, 1, 0)
    first = cid * CHUNKS_PER_CORE + sid * FULL_PER_TILE + jnp.minimum(sid, REM)

    rsl = pl.ds(sid * ROWS_PER_TILE, ROWS_PER_TILE)
    rsl_last = pl.ds(15 * ROWS_PER_TILE, LAST_ROWS)

    def init_write(src_full, src_last):
        @pl.when(sid < 15)
        def _():
            pltpu.sync_copy(src_full, acc_sh.at[rsl])

        @pl.when(sid == 15)
        def _():
            pltpu.sync_copy(src_last, acc_sh.at[rsl_last])

    @pl.when(cid == 0)
    def _():
        init_write(h_hbm.at[rsl], h_hbm.at[rsl_last])

    @pl.when(cid == 1)
    def _():
        init_write(zeros_hbm.at[pl.ds(0, ROWS_PER_TILE)],
                   zeros_hbm.at[pl.ds(0, LAST_ROWS)])

    plsc.subcore_barrier()

    def istart(i, ib):
        base = (first + i) * C
        pltpu.async_copy(src_hbm.at[pl.ds(base, C)], sidx_v.at[ib],
                         isem.at[ib])
        pltpu.async_copy(dst_hbm.at[pl.ds(base, C)], didx_v.at[ib],
                         isem.at[ib])

    def iwait(ib):
        pltpu.make_async_copy(src_hbm.at[pl.ds(0, C)], sidx_v.at[ib],
                              isem.at[ib]).wait()
        pltpu.make_async_copy(dst_hbm.at[pl.ds(0, C)], didx_v.at[ib],
                              isem.at[ib]).wait()

    def gather_start(ib, b):
        pltpu.async_copy(h_hbm.at[sidx_v.at[ib]], rows_v.at[b], gsem.at[b])

    def gather_wait(b):
        pltpu.make_async_copy(h_hbm.at[sidx_v.at[0]], rows_v.at[b],
                              gsem.at[b]).wait()

    def scatter_start(ib, b):
        pltpu.async_copy(rows_v.at[b], acc_sh.at[didx_v.at[ib]],
                         ssem.at[b], add=True)

    def scatter_wait(b):
        pltpu.make_async_copy(rows_v.at[b], acc_sh.at[didx_v.at[0]],
                              ssem.at[b]).wait()

    for j in range(NIB - 1):
        istart(j, j)
    iwait(0)
    gather_start(0, 0)

    def body(jj, carry):
        for k in range(4):
            s = 1 + jj * 4 + k
            b = s % 2
            o = 1 - b
            ib = s % 4
            ibp = (s - 1) % 4
            ibn = (s + 2) % 4

            @pl.when(s <= n_i - 1)
            def _():
                @pl.when(s >= 2)
                def _():
                    scatter_wait(b)

                @pl.when(s + 2 <= n_i - 1)
                def _():
                    istart(s + 2, ibn)

                iwait(ib)
                gather_start(ib, b)

            @pl.when(s <= n_i)
            def _():
                gather_wait(o)
                scatter_start(ibp, o)
        return carry

    lax.fori_loop(0, (FULL_PER_TILE + 1 + 3) // 4, body, 0)

    scatter_wait(0)
    scatter_wait(1)

    plsc.subcore_barrier()

    def write_to(o_hbm):
        @pl.when(sid < 15)
        def _():
            pltpu.sync_copy(acc_sh.at[rsl], o_hbm.at[rsl])

        @pl.when(sid == 15)
        def _():
            pltpu.sync_copy(acc_sh.at[rsl_last], o_hbm.at[rsl_last])

    @pl.when(cid == 0)
    def _():
        write_to(o0_hbm)

    @pl.when(cid == 1)
    def _():
        write_to(o1_hbm)


ROW_BLK = 1000
GRID = N // ROW_BLK

_blk_spec = pl.BlockSpec((ROW_BLK, D), lambda i: (i, 0))

_SELU_ALPHA = 1.6732632423543772
_SELU_SCALE = 1.0507009873554805


def _add2_body(p0_ref, p1_ref, o_ref):
    o_ref[...] = p0_ref[...] + p1_ref[...]


def _add2(p0, p1):
    return pl.pallas_call(
        _add2_body,
        grid=(GRID,),
        in_specs=[_blk_spec, _blk_spec],
        out_specs=_blk_spec,
        out_shape=_full,
    )(p0, p1)


def _mlp_body(q0_ref, q1_ref, w_ref, b_ref, o_ref):
    h2 = q0_ref[...] + q1_ref[...]
    z = jnp.dot(h2, w_ref[...], preferred_element_type=jnp.float32)
    z = z + b_ref[...]
    o_ref[...] = _SELU_SCALE * jnp.where(
        z > 0, z, _SELU_ALPHA * (jnp.exp(z) - 1.0))


def _mlp(q0, q1, W1, b1):
    return pl.pallas_call(
        _mlp_body,
        grid=(GRID,),
        in_specs=[
            _blk_spec, _blk_spec,
            pl.BlockSpec((D, D), lambda i: (0, 0)),
            pl.BlockSpec((1, D), lambda i: (0, 0)),
        ],
        out_specs=_blk_spec,
        out_shape=_full,
    )(q0, q1, W1, b1.reshape(1, D))


def _out_body(r0_ref, r1_ref, w_ref, b_ref, o_ref):
    g = r0_ref[...] + r1_ref[...]
    g = jnp.dot(g, w_ref[...], preferred_element_type=jnp.float32)
    g = g + b_ref[...]
    m = jnp.max(g, axis=1, keepdims=True)
    e = g - m
    lse = jnp.log(jnp.sum(jnp.exp(e), axis=1, keepdims=True))
    o_ref[...] = e - lse


def _outp(r0, r1, W2, b2):
    odim = W2.shape[1]
    return pl.pallas_call(
        _out_body,
        grid=(GRID,),
        in_specs=[
            _blk_spec, _blk_spec,
            pl.BlockSpec((D, odim), lambda i: (0, 0)),
            pl.BlockSpec((1, odim), lambda i: (0, 0)),
        ],
        out_specs=pl.BlockSpec((ROW_BLK, odim), lambda i: (i, 0)),
        out_shape=jax.ShapeDtypeStruct((N, odim), jnp.float32),
    )(r0, r1, W2, b2.reshape(1, odim))


def kernel(x, edge_index, W1, b1, W2, b2):
    src = edge_index[0].astype(jnp.int32)
    dst = edge_index[1].astype(jnp.int32)
    zeros = jnp.zeros((ROWS_PER_TILE, D), jnp.float32)

    p0, p1 = _prop(x, src, dst, zeros)
    h1 = _add2(p0, p1)
    q0, q1 = _prop(h1, src, dst, zeros)
    h = _mlp(q0, q1, W1, b1)
    r0, r1 = _prop(h, src, dst, zeros)
    return _outp(r0, r1, W2, b2)


# speedup vs baseline: 12.1734x; 1.0657x over previous
import functools

import jax
import jax.numpy as jnp
from jax import lax
from jax.experimental import pallas as pl
from jax.experimental.pallas import tpu as pltpu
from jax.experimental.pallas import tpu_sc as plsc

N = 10000
E = 320000
D = 128
NC, NS = 2, 16
ROWS_PER_TILE = 632
LAST_ROWS = N - 15 * ROWS_PER_TILE
C = 128
CHUNKS = E // C
CHUNKS_PER_CORE = CHUNKS // NC
FULL_PER_TILE = CHUNKS_PER_CORE // NS
REM = CHUNKS_PER_CORE - FULL_PER_TILE * NS
NIB = 4
NRB = 3
UNROLL = 12

_mesh = plsc.VectorSubcoreMesh(core_axis_name="c", subcore_axis_name="s")

_full = jax.ShapeDtypeStruct((N, D), jnp.float32)


@functools.partial(
    pl.kernel,
    mesh=_mesh,
    out_type=(_full, _full),
    scratch_types=[
        pltpu.VMEM((NIB, C), jnp.int32),
        pltpu.VMEM((NIB, C), jnp.int32),
        pltpu.VMEM((NRB, C, D), jnp.float32),
        pltpu.VMEM_SHARED((N, D), jnp.float32),
        pltpu.SemaphoreType.DMA((NIB,)),
        pltpu.SemaphoreType.DMA((NRB,)),
        pltpu.SemaphoreType.DMA((NRB,)),
    ],
)
def _prop(h_hbm, src_hbm, dst_hbm, zeros_hbm, o0_hbm, o1_hbm,
          sidx_v, didx_v, rows_v, acc_sh, isem, gsem, ssem):
    cid = lax.axis_index("c")
    sid = lax.axis_index("s")

    n_i = FULL_PER_TILE + jnp.where(sid < REM, 1, 0)
    first = cid * CHUNKS_PER_CORE + sid * FULL_PER_TILE + jnp.minimum(sid, REM)

    rsl = pl.ds(sid * ROWS_PER_TILE, ROWS_PER_TILE)
    rsl_last = pl.ds(15 * ROWS_PER_TILE, LAST_ROWS)

    def init_write(src_full, src_last):
        @pl.when(sid < 15)
        def _():
            pltpu.sync_copy(src_full, acc_sh.at[rsl])

        @pl.when(sid == 15)
        def _():
            pltpu.sync_copy(src_last, acc_sh.at[rsl_last])

    @pl.when(cid == 0)
    def _():
        init_write(h_hbm.at[rsl], h_hbm.at[rsl_last])

    @pl.when(cid == 1)
    def _():
        init_write(zeros_hbm.at[pl.ds(0, ROWS_PER_TILE)],
                   zeros_hbm.at[pl.ds(0, LAST_ROWS)])

    plsc.subcore_barrier()

    def istart(i, ib):
        base = (first + i) * C
        pltpu.async_copy(src_hbm.at[pl.ds(base, C)], sidx_v.at[ib],
                         isem.at[ib])
        pltpu.async_copy(dst_hbm.at[pl.ds(base, C)], didx_v.at[ib],
                         isem.at[ib])

    def iwait(ib):
        pltpu.make_async_copy(src_hbm.at[pl.ds(0, C)], sidx_v.at[ib],
                              isem.at[ib]).wait()
        pltpu.make_async_copy(dst_hbm.at[pl.ds(0, C)], didx_v.at[ib],
                              isem.at[ib]).wait()

    def gather_start(ib, b):
        pltpu.async_copy(h_hbm.at[sidx_v.at[ib]], rows_v.at[b], gsem.at[b])

    def gather_wait(b):
        pltpu.make_async_copy(h_hbm.at[sidx_v.at[0]], rows_v.at[b],
                              gsem.at[b]).wait()

    def scatter_start(ib, b):
        pltpu.async_copy(rows_v.at[b], acc_sh.at[didx_v.at[ib]],
                         ssem.at[b], add=True)

    def scatter_wait(b):
        pltpu.make_async_copy(rows_v.at[b], acc_sh.at[didx_v.at[0]],
                              ssem.at[b]).wait()

    for j in range(NIB):
        istart(j, j)
    iwait(0)
    gather_start(0, 0)

    def body(jj, carry):
        for k in range(UNROLL):
            s = 1 + jj * UNROLL + k
            b = s % NRB
            o = (s - 1) % NRB
            ib = s % NIB
            ibp = (s - 1) % NIB
            ibn = (s + 1) % NIB

            @pl.when(s <= n_i - 1)
            def _():
                @pl.when(s >= NRB)
                def _():
                    scatter_wait(b)

                @pl.when(jnp.logical_and(s + 1 <= n_i - 1, s >= NIB - 1))
                def _():
                    istart(s + 1, ibn)

                iwait(ib)
                gather_start(ib, b)

            @pl.when(s <= n_i)
            def _():
                gather_wait(o)
                scatter_start(ibp, o)
        return carry

    lax.fori_loop(0, (FULL_PER_TILE + 1 + UNROLL - 1) // UNROLL, body, 0)

    for b in range(NRB):
        scatter_wait(b)

    plsc.subcore_barrier()

    def write_to(o_hbm):
        @pl.when(sid < 15)
        def _():
            pltpu.sync_copy(acc_sh.at[rsl], o_hbm.at[rsl])

        @pl.when(sid == 15)
        def _():
            pltpu.sync_copy(acc_sh.at[rsl_last], o_hbm.at[rsl_last])

    @pl.when(cid == 0)
    def _():
        write_to(o0_hbm)

    @pl.when(cid == 1)
    def _():
        write_to(o1_hbm)


ROW_BLK = 1000
GRID = N // ROW_BLK

_blk_spec = pl.BlockSpec((ROW_BLK, D), lambda i: (i, 0))

_SELU_ALPHA = 1.6732632423543772
_SELU_SCALE = 1.0507009873554805


def _add2_body(p0_ref, p1_ref, o_ref):
    o_ref[...] = p0_ref[...] + p1_ref[...]


def _add2(p0, p1):
    return pl.pallas_call(
        _add2_body,
        grid=(GRID,),
        in_specs=[_blk_spec, _blk_spec],
        out_specs=_blk_spec,
        out_shape=_full,
    )(p0, p1)


def _mlp_body(q0_ref, q1_ref, w_ref, b_ref, o_ref):
    h2 = q0_ref[...] + q1_ref[...]
    z = jnp.dot(h2, w_ref[...], preferred_element_type=jnp.float32)
    z = z + b_ref[...]
    o_ref[...] = _SELU_SCALE * jnp.where(
        z > 0, z, _SELU_ALPHA * (jnp.exp(z) - 1.0))


def _mlp(q0, q1, W1, b1):
    return pl.pallas_call(
        _mlp_body,
        grid=(GRID,),
        in_specs=[
            _blk_spec, _blk_spec,
            pl.BlockSpec((D, D), lambda i: (0, 0)),
            pl.BlockSpec((1, D), lambda i: (0, 0)),
        ],
        out_specs=_blk_spec,
        out_shape=_full,
    )(q0, q1, W1, b1.reshape(1, D))


def _out_body(r0_ref, r1_ref, w_ref, b_ref, o_ref):
    g = r0_ref[...] + r1_ref[...]
    g = jnp.dot(g, w_ref[...], preferred_element_type=jnp.float32)
    g = g + b_ref[...]
    m = jnp.max(g, axis=1, keepdims=True)
    e = g - m
    lse = jnp.log(jnp.sum(jnp.exp(e), axis=1, keepdims=True))
    o_ref[...] = e - lse


def _outp(r0, r1, W2, b2):
    odim = W2.shape[1]
    return pl.pallas_call(
        _out_body,
        grid=(GRID,),
        in_specs=[
            _blk_spec, _blk_spec,
            pl.BlockSpec((D, odim), lambda i: (0, 0)),
            pl.BlockSpec((1, odim), lambda i: (0, 0)),
        ],
        out_specs=pl.BlockSpec((ROW_BLK, odim), lambda i: (i, 0)),
        out_shape=jax.ShapeDtypeStruct((N, odim), jnp.float32),
    )(r0, r1, W2, b2.reshape(1, odim))


def kernel(x, edge_index, W1, b1, W2, b2):
    src = edge_index[0].astype(jnp.int32)
    dst = edge_index[1].astype(jnp.int32)
    zeros = jnp.zeros((ROWS_PER_TILE, D), jnp.float32)

    p0, p1 = _prop(x, src, dst, zeros)
    h1 = _add2(p0, p1)
    q0, q1 = _prop(h1, src, dst, zeros)
    h = _mlp(q0, q1, W1, b1)
    r0, r1 = _prop(h, src, dst, zeros)
    return _outp(r0, r1, W2, b2)


# speedup vs baseline: 13.2565x; 1.0890x over previous
import functools

import jax
import jax.numpy as jnp
from jax import lax
from jax.experimental import pallas as pl
from jax.experimental.pallas import tpu as pltpu
from jax.experimental.pallas import tpu_sc as plsc

N = 10000
E = 320000
D = 128
DO = 64
NC, NS = 2, 16
ROWS_PER_TILE = 632
LAST_ROWS = N - 15 * ROWS_PER_TILE
C = 128
CHUNKS = E // C
CHUNKS_PER_CORE = CHUNKS // NC
FULL_PER_TILE = CHUNKS_PER_CORE // NS
REM = CHUNKS_PER_CORE - FULL_PER_TILE * NS
NIB = 4
NRB = 3
UNROLL = 12

_mesh = plsc.VectorSubcoreMesh(core_axis_name="c", subcore_axis_name="s")


def _make_prop(width, tc_tiling):
    shape = jax.ShapeDtypeStruct((N, width), jnp.float32)

    @functools.partial(
        pl.kernel,
        mesh=_mesh,
        out_type=(shape, shape),
        compiler_params=pltpu.CompilerParams(use_tc_tiling_on_sc=tc_tiling),
        scratch_types=[
            pltpu.VMEM((NIB, C), jnp.int32),
            pltpu.VMEM((NIB, C), jnp.int32),
            pltpu.VMEM((NRB, C, width), jnp.float32),
            pltpu.VMEM_SHARED((N, width), jnp.float32),
            pltpu.SemaphoreType.DMA((NIB,)),
            pltpu.SemaphoreType.DMA((NRB,)),
            pltpu.SemaphoreType.DMA((NRB,)),
        ],
    )
    def prop(h_hbm, src_hbm, dst_hbm, zeros_hbm, o0_hbm, o1_hbm,
             sidx_v, didx_v, rows_v, acc_sh, isem, gsem, ssem):
        cid = lax.axis_index("c")
        sid = lax.axis_index("s")

        n_i = FULL_PER_TILE + jnp.where(sid < REM, 1, 0)
        first = (cid * CHUNKS_PER_CORE + sid * FULL_PER_TILE
                 + jnp.minimum(sid, REM))

        def istart(i, ib):
            base = (first + i) * C
            pltpu.async_copy(src_hbm.at[pl.ds(base, C)], sidx_v.at[ib],
                             isem.at[ib])
            pltpu.async_copy(dst_hbm.at[pl.ds(base, C)], didx_v.at[ib],
                             isem.at[ib])

        def iwait(ib):
            pltpu.make_async_copy(src_hbm.at[pl.ds(0, C)], sidx_v.at[ib],
                                  isem.at[ib]).wait()
            pltpu.make_async_copy(dst_hbm.at[pl.ds(0, C)], didx_v.at[ib],
                                  isem.at[ib]).wait()

        def gather_start(ib, b):
            pltpu.async_copy(h_hbm.at[sidx_v.at[ib]], rows_v.at[b],
                             gsem.at[b])

        def gather_wait(b):
            pltpu.make_async_copy(h_hbm.at[sidx_v.at[0]], rows_v.at[b],
                                  gsem.at[b]).wait()

        def scatter_start(ib, b):
            pltpu.async_copy(rows_v.at[b], acc_sh.at[didx_v.at[ib]],
                             ssem.at[b], add=True)

        def scatter_wait(b):
            pltpu.make_async_copy(rows_v.at[b], acc_sh.at[didx_v.at[0]],
                                  ssem.at[b]).wait()

        for j in range(NIB):
            istart(j, j)
        iwait(0)
        gather_start(0, 0)

        rsl = pl.ds(sid * ROWS_PER_TILE, ROWS_PER_TILE)
        rsl_last = pl.ds(15 * ROWS_PER_TILE, LAST_ROWS)

        def init_write(src_full, src_last):
            @pl.when(sid < 15)
            def _():
                pltpu.sync_copy(src_full, acc_sh.at[rsl])

            @pl.when(sid == 15)
            def _():
                pltpu.sync_copy(src_last, acc_sh.at[rsl_last])

        @pl.when(cid == 0)
        def _():
            init_write(h_hbm.at[rsl], h_hbm.at[rsl_last])

        @pl.when(cid == 1)
        def _():
            init_write(zeros_hbm.at[pl.ds(0, ROWS_PER_TILE)],
                       zeros_hbm.at[pl.ds(0, LAST_ROWS)])

        plsc.subcore_barrier()

        def body(jj, carry):
            for k in range(UNROLL):
                s = 1 + jj * UNROLL + k
                b = s % NRB
                o = (s - 1) % NRB
                ib = s % NIB
                ibp = (s - 1) % NIB
                ibn = (s + 1) % NIB

                @pl.when(s <= n_i - 1)
                def _():
                    @pl.when(s >= NRB)
                    def _():
                        scatter_wait(b)

                    @pl.when(jnp.logical_and(s + 1 <= n_i - 1,
                                             s >= NIB - 1))
                    def _():
                        istart(s + 1, ibn)

                    iwait(ib)
                    gather_start(ib, b)

                @pl.when(s <= n_i)
                def _():
                    gather_wait(o)
                    scatter_start(ibp, o)
            return carry

        lax.fori_loop(0, (FULL_PER_TILE + 1 + UNROLL - 1) // UNROLL, body, 0)

        for b in range(NRB):
            scatter_wait(b)

        plsc.subcore_barrier()

        def write_to(o_hbm):
            @pl.when(sid < 15)
            def _():
                pltpu.sync_copy(acc_sh.at[rsl], o_hbm.at[rsl])

            @pl.when(sid == 15)
            def _():
                pltpu.sync_copy(acc_sh.at[rsl_last], o_hbm.at[rsl_last])

        @pl.when(cid == 0)
        def _():
            write_to(o0_hbm)

        @pl.when(cid == 1)
        def _():
            write_to(o1_hbm)

    return prop


_prop = _make_prop(D, True)
_prop_out = _make_prop(DO, False)


ROW_BLK = 1000
GRID = N // ROW_BLK

_blk_spec = pl.BlockSpec((ROW_BLK, D), lambda i: (i, 0))
_out_spec = pl.BlockSpec((ROW_BLK, DO), lambda i: (i, 0))
_full = jax.ShapeDtypeStruct((N, D), jnp.float32)
_half = jax.ShapeDtypeStruct((N, DO), jnp.float32)

_SELU_ALPHA = 1.6732632423543772
_SELU_SCALE = 1.0507009873554805


def _add2_body(p0_ref, p1_ref, o_ref):
    o_ref[...] = p0_ref[...] + p1_ref[...]


def _add2(p0, p1):
    return pl.pallas_call(
        _add2_body,
        grid=(GRID,),
        in_specs=[_blk_spec, _blk_spec],
        out_specs=_blk_spec,
        out_shape=_full,
    )(p0, p1)


def _mlp_body(q0_ref, q1_ref, w1_ref, b1_ref, w2_ref, o_ref):
    h2 = q0_ref[...] + q1_ref[...]
    z = jnp.dot(h2, w1_ref[...], preferred_element_type=jnp.float32)
    z = z + b1_ref[...]
    h = _SELU_SCALE * jnp.where(z > 0, z, _SELU_ALPHA * (jnp.exp(z) - 1.0))
    o_ref[...] = jnp.dot(h, w2_ref[...], preferred_element_type=jnp.float32)


def _mlp(q0, q1, W1, b1, W2):
    return pl.pallas_call(
        _mlp_body,
        grid=(GRID,),
        in_specs=[
            _blk_spec, _blk_spec,
            pl.BlockSpec((D, D), lambda i: (0, 0)),
            pl.BlockSpec((1, D), lambda i: (0, 0)),
            pl.BlockSpec((D, DO), lambda i: (0, 0)),
        ],
        out_specs=_out_spec,
        out_shape=_half,
    )(q0, q1, W1, b1.reshape(1, D), W2)


def _out_body(r0_ref, r1_ref, b_ref, o_ref):
    g = r0_ref[...] + r1_ref[...] + b_ref[...]
    m = jnp.max(g, axis=1, keepdims=True)
    e = g - m
    lse = jnp.log(jnp.sum(jnp.exp(e), axis=1, keepdims=True))
    o_ref[...] = e - lse


def _outp(r0, r1, b2):
    return pl.pallas_call(
        _out_body,
        grid=(GRID,),
        in_specs=[
            _out_spec, _out_spec,
            pl.BlockSpec((1, DO), lambda i: (0, 0)),
        ],
        out_specs=_out_spec,
        out_shape=_half,
    )(r0, r1, b2.reshape(1, DO))


def kernel(x, edge_index, W1, b1, W2, b2):
    src = edge_index[0].astype(jnp.int32)
    dst = edge_index[1].astype(jnp.int32)
    zeros = jnp.zeros((ROWS_PER_TILE, D), jnp.float32)
    zeros_o = jnp.zeros((ROWS_PER_TILE, DO), jnp.float32)

    p0, p1 = _prop(x, src, dst, zeros)
    h1 = _add2(p0, p1)
    q0, q1 = _prop(h1, src, dst, zeros)
    t = _mlp(q0, q1, W1, b1, W2)
    r0, r1 = _prop_out(t, src, dst, zeros_o)
    return _outp(r0, r1, b2)
